# baseline (speedup 1.0000x reference)
import functools

import jax
import jax.numpy as jnp
from jax import lax
from jax.experimental import pallas as pl
from jax.experimental.pallas import tpu as pltpu
from jax.experimental.pallas import tpu_sc as plsc

_INFO = plsc.get_sparse_core_info()
_NC, _NS = _INFO.num_cores, _INFO.num_subcores
_NW = _NC * _NS

_ROWS, _D = 100000, 128
_B0, _B1 = 4096, 50
_N = _B0 * _B1
_CHUNK = 64
_NCHUNKS = _N // _CHUNK
_CPW = _NCHUNKS // _NW
_NBUF = 10
_NG = _CPW // _NBUF


@functools.partial(
    pl.kernel,
    out_type=jax.ShapeDtypeStruct((_N, _D), jnp.float32),
    mesh=plsc.VectorSubcoreMesh(core_axis_name="c", subcore_axis_name="s"),
    scratch_types=[
        pltpu.VMEM((_CPW, _CHUNK), jnp.int32),
        pltpu.VMEM((_NBUF, _CHUNK, _D), jnp.float32),
    ]
    + [pltpu.SemaphoreType.DMA] * (2 * _NBUF),
)
def _gather_sc(table_hbm, idx_hbm, out_hbm, idx_v, rows_v, *sems):
    gsems, wsems = sems[:_NBUF], sems[_NBUF:]
    wid = lax.axis_index("s") * _NC + lax.axis_index("c")
    base = wid * _CPW
    pltpu.sync_copy(idx_hbm.at[wid], idx_v)

    def out_slice(j):
        off = pl.multiple_of((base + j) * _CHUNK, _CHUNK)
        return out_hbm.at[pl.ds(off, _CHUNK)]

    for b in range(_NBUF):
        pltpu.async_copy(table_hbm.at[idx_v.at[b]], rows_v.at[b], gsems[b])

    def group(g, carry):
        for b in range(_NBUF):
            j = g * _NBUF + b
            pltpu.make_async_copy(
                table_hbm.at[idx_v.at[j]], rows_v.at[b], gsems[b]).wait()
            pltpu.async_copy(rows_v.at[b], out_slice(j), wsems[b])
        for b in range(_NBUF):
            jn = (g + 1) * _NBUF + b
            pltpu.make_async_copy(
                rows_v.at[b], out_slice(jn - _NBUF), wsems[b]).wait()
            pltpu.async_copy(table_hbm.at[idx_v.at[jn]], rows_v.at[b], gsems[b])
        return carry

    lax.fori_loop(0, _NG - 1, group, 0)

    last = (_NG - 1) * _NBUF
    for b in range(_NBUF):
        j = last + b
        pltpu.make_async_copy(
            table_hbm.at[idx_v.at[j]], rows_v.at[b], gsems[b]).wait()
        pltpu.async_copy(rows_v.at[b], out_slice(j), wsems[b])
    for b in range(_NBUF):
        pltpu.make_async_copy(
            rows_v.at[b], out_slice(last + b), wsems[b]).wait()


def kernel(tensor, indexes):
    idx_t = indexes.astype(jnp.int32).T.reshape(_NW, _CPW, _CHUNK)
    out = _gather_sc(tensor, idx_t)
    return out.reshape(_B1, _B0, _D).transpose(1, 0, 2)


# speedup vs baseline: 1.0074x; 1.0074x over previous
import functools

import jax
import jax.numpy as jnp
from jax import lax
from jax.experimental import pallas as pl
from jax.experimental.pallas import tpu as pltpu
from jax.experimental.pallas import tpu_sc as plsc

_INFO = plsc.get_sparse_core_info()
_NC, _NS = _INFO.num_cores, _INFO.num_subcores
_NW = _NC * _NS

_ROWS, _D = 100000, 128
_B0, _B1 = 4096, 50
_N = _B0 * _B1
_CHUNK = 64
_NCHUNKS = _N // _CHUNK
_CPW = _NCHUNKS // _NW
_NBUF = 5
_NG = _CPW // _NBUF


@functools.partial(
    pl.kernel,
    out_type=jax.ShapeDtypeStruct((_N, _D), jnp.float32),
    mesh=plsc.VectorSubcoreMesh(core_axis_name="c", subcore_axis_name="s"),
    scratch_types=[
        pltpu.VMEM((_CPW, _CHUNK), jnp.int32),
        pltpu.VMEM((_NBUF, _CHUNK, _D), jnp.float32),
    ]
    + [pltpu.SemaphoreType.DMA] * (2 * _NBUF),
)
def _gather_sc(table_hbm, idx_hbm, out_hbm, idx_v, rows_v, *sems):
    gsems, wsems = sems[:_NBUF], sems[_NBUF:]
    wid = lax.axis_index("s") * _NC + lax.axis_index("c")
    base = wid * _CPW
    pltpu.sync_copy(idx_hbm.at[wid], idx_v)

    def out_slice(j):
        off = pl.multiple_of((base + j) * _CHUNK, _CHUNK)
        return out_hbm.at[pl.ds(off, _CHUNK)]

    for b in range(_NBUF):
        pltpu.async_copy(table_hbm.at[idx_v.at[b]], rows_v.at[b], gsems[b])

    def group(g, carry):
        for b in range(_NBUF):
            j = g * _NBUF + b
            pltpu.make_async_copy(
                table_hbm.at[idx_v.at[j]], rows_v.at[b], gsems[b]).wait()
            pltpu.async_copy(rows_v.at[b], out_slice(j), wsems[b])
        for b in range(_NBUF):
            jn = (g + 1) * _NBUF + b
            pltpu.make_async_copy(
                rows_v.at[b], out_slice(jn - _NBUF), wsems[b]).wait()
            pltpu.async_copy(table_hbm.at[idx_v.at[jn]], rows_v.at[b], gsems[b])
        return carry

    lax.fori_loop(0, _NG - 1, group, 0)

    last = (_NG - 1) * _NBUF
    for b in range(_NBUF):
        j = last + b
        pltpu.make_async_copy(
            table_hbm.at[idx_v.at[j]], rows_v.at[b], gsems[b]).wait()
        pltpu.async_copy(rows_v.at[b], out_slice(j), wsems[b])
    for b in range(_NBUF):
        pltpu.make_async_copy(
            rows_v.at[b], out_slice(last + b), wsems[b]).wait()


def kernel(tensor, indexes):
    idx_t = indexes.astype(jnp.int32).T.reshape(_NW, _CPW, _CHUNK)
    out = _gather_sc(tensor, idx_t)
    return out.reshape(_B1, _B0, _D).transpose(1, 0, 2)
